# baseline (speedup 1.0000x reference)
import functools

import jax
import jax.numpy as jnp
from jax import lax
from jax.experimental import pallas as pl
from jax.experimental.pallas import tpu as pltpu
from jax.experimental.pallas import tpu_sc as plsc

_NC = 2
_NS = 16
_NW = _NC * _NS

_NBUF = 5


@jax.jit
def _sc_gather(data, idx_t):
    n_j, b = idx_t.shape
    d = data.shape[1]
    bw = b // _NW
    assert n_j % _NBUF == 0 and n_j >= 2 * _NBUF
    mesh = plsc.VectorSubcoreMesh(
        core_axis_name="c", subcore_axis_name="s",
        num_cores=_NC, num_subcores=_NS,
    )

    @functools.partial(
        pl.kernel,
        out_type=jax.ShapeDtypeStruct((n_j, b, d), data.dtype),
        mesh=mesh,
        scratch_types=[
            pltpu.VMEM((n_j, bw), jnp.int32),
            pltpu.VMEM((_NBUF, bw, d), data.dtype),
            pltpu.SemaphoreType.DMA,
            pltpu.SemaphoreType.DMA((_NBUF,)),
            pltpu.SemaphoreType.DMA((_NBUF,)),
        ],
        compiler_params=pltpu.CompilerParams(use_tc_tiling_on_sc=False),
    )
    def k(table_hbm, idx_hbm, out_hbm, idx_v, rows_v, isem, gsem, wsem):
        wid = lax.axis_index("s") * _NC + lax.axis_index("c")
        b0 = wid * bw

        pltpu.async_copy(
            idx_hbm.at[:, pl.ds(b0, bw)], idx_v, isem).wait()

        def start_gather(g, slot):
            return pltpu.async_copy(
                table_hbm.at[idx_v.at[g]], rows_v.at[slot], gsem.at[slot])

        def start_write(g, slot):
            return pltpu.async_copy(
                rows_v.at[slot], out_hbm.at[g, pl.ds(b0, bw)],
                wsem.at[slot])

        def wait_gather(slot):
            pltpu.make_async_copy(
                table_hbm.at[idx_v.at[0]], rows_v.at[slot],
                gsem.at[slot]).wait()

        def wait_write(slot):
            pltpu.make_async_copy(
                rows_v.at[slot], out_hbm.at[0, pl.ds(b0, bw)],
                wsem.at[slot]).wait()

        start_gather(0, 0)
        for g in range(1, _NBUF):
            start_gather(g, g)
            wait_gather(g - 1)
            start_write(g - 1, g - 1)

        def outer(t, carry):
            g0 = _NBUF + t * _NBUF
            for s in range(_NBUF):
                g = g0 + s
                wait_write(s)
                start_gather(g, s)
                sp = (s - 1) % _NBUF
                wait_gather(sp)
                start_write(g - 1, sp)
            return carry

        lax.fori_loop(0, n_j // _NBUF - 1, outer, 0)

        last_s = (n_j - 1) % _NBUF
        wait_gather(last_s)
        start_write(n_j - 1, last_s)
        for s in range(_NBUF):
            wait_write(s)

    return k(data, idx_t)


def kernel(data, indices, axis):
    del axis
    d = data.shape[1]
    table = jnp.pad(data, ((0, 0), (0, 128 - d))).reshape(-1, d)
    idx_t = indices.T * (128 // d)
    n_j = idx_t.shape[0]
    o1 = _sc_gather(table, idx_t[: n_j // 2])
    o2 = _sc_gather(table, idx_t[n_j // 2:])
    return jnp.concatenate([o1, o2], axis=0).transpose(1, 0, 2)


# speedup vs baseline: 1.0368x; 1.0368x over previous
import functools

import jax
import jax.numpy as jnp
from jax import lax
from jax.experimental import pallas as pl
from jax.experimental.pallas import tpu as pltpu
from jax.experimental.pallas import tpu_sc as plsc

_NC = 2
_NS = 16
_NW = _NC * _NS

_NBUF = 5


@jax.jit
def _sc_gather(data, idx_t):
    n_j, b = idx_t.shape
    d = data.shape[1]
    bw = b // _NW
    assert n_j % _NBUF == 0 and n_j >= 2 * _NBUF
    mesh = plsc.VectorSubcoreMesh(
        core_axis_name="c", subcore_axis_name="s",
        num_cores=_NC, num_subcores=_NS,
    )

    @functools.partial(
        pl.kernel,
        out_type=jax.ShapeDtypeStruct((n_j, b, d), data.dtype),
        mesh=mesh,
        scratch_types=[
            pltpu.VMEM((n_j, bw), jnp.int32),
            pltpu.VMEM((_NBUF, bw, d), data.dtype),
            pltpu.SemaphoreType.DMA,
            pltpu.SemaphoreType.DMA((_NBUF,)),
            pltpu.SemaphoreType.DMA((_NBUF,)),
        ],
        compiler_params=pltpu.CompilerParams(use_tc_tiling_on_sc=False),
    )
    def k(table_hbm, idx_hbm, out_hbm, idx_v, rows_v, isem, gsem, wsem):
        wid = lax.axis_index("s") * _NC + lax.axis_index("c")
        b0 = wid * bw

        pltpu.async_copy(
            idx_hbm.at[:, pl.ds(b0, bw)], idx_v, isem).wait()

        def start_gather(g, slot):
            return pltpu.async_copy(
                table_hbm.at[idx_v.at[g]], rows_v.at[slot], gsem.at[slot])

        def start_write(g, slot):
            return pltpu.async_copy(
                rows_v.at[slot], out_hbm.at[g, pl.ds(b0, bw)],
                wsem.at[slot])

        def wait_gather(slot):
            pltpu.make_async_copy(
                table_hbm.at[idx_v.at[0]], rows_v.at[slot],
                gsem.at[slot]).wait()

        def wait_write(slot):
            pltpu.make_async_copy(
                rows_v.at[slot], out_hbm.at[0, pl.ds(b0, bw)],
                wsem.at[slot]).wait()

        start_gather(0, 0)
        for g in range(1, _NBUF):
            start_gather(g, g)
            wait_gather(g - 1)
            start_write(g - 1, g - 1)

        def outer(t, carry):
            g0 = _NBUF + t * _NBUF
            for s in range(_NBUF):
                g = g0 + s
                wait_write(s)
                start_gather(g, s)
                sp = (s - 1) % _NBUF
                wait_gather(sp)
                start_write(g - 1, sp)
            return carry

        lax.fori_loop(0, n_j // _NBUF - 1, outer, 0)

        last_s = (n_j - 1) % _NBUF
        wait_gather(last_s)
        start_write(n_j - 1, last_s)
        for s in range(_NBUF):
            wait_write(s)

    return k(data, idx_t)


def kernel(data, indices, axis):
    del axis
    d = data.shape[1]
    table = jnp.pad(data, ((0, 0), (0, 128 - d))).reshape(-1, d)
    out = _sc_gather(table, indices.T * (128 // d))
    return out.transpose(1, 0, 2)
